# baseline (speedup 1.0000x reference)
import jax
import jax.numpy as jnp
from jax.experimental import pallas as pl
from jax.experimental.pallas import tpu as pltpu

B = 128
V = 100000
NC = 782
NCM = 781
NG = 7
WIN = 962
NWIN = 13
NT = V // 8
RB = 32
NEG_INF = float("-inf")


def _max_body(x_ref, m_ref, macc):
    w = pl.program_id(0)

    @pl.when(w == 0)
    def _():
        macc[...] = jnp.full((8, 128), NEG_INF, jnp.float32)

    valid = 12500 - WIN * w

    def step(t, acc):
        for u in range(13):
            g = t * 13 + u
            sl = x_ref[pl.ds(g * 8, 8), :]
            sl = jnp.where(g < valid, sl, NEG_INF)
            acc = jnp.maximum(acc, sl)
        return acc

    macc[...] = jax.lax.fori_loop(0, WIN // 13, step, macc[...])

    @pl.when(w == NWIN - 1)
    def _():
        m_ref[...] = jnp.max(macc[...], axis=0, keepdims=True)


def _z_body(x_ref, m_ref, z_ref, tot):
    w = pl.program_id(0)

    @pl.when(w == 0)
    def _():
        tot[...] = jnp.zeros((1, 128), jnp.float32)

    m = m_ref[...]
    valid = 12500 - WIN * w

    def step(t, acc):
        for u in range(13):
            g = t * 13 + u
            sl = x_ref[pl.ds(g * 8, 8), :]
            e = jnp.exp(sl - m)
            e = jnp.where(g < valid, e, jnp.float32(0.0))
            acc = acc + e
        return acc

    acc = jax.lax.fori_loop(0, WIN // 13, step,
                            jnp.zeros((8, 128), jnp.float32))
    b4 = acc[0:4, :] + acc[4:8, :]
    b2 = b4[0:2, :] + b4[2:4, :]
    b1 = b2[0:1, :] + b2[1:2, :]
    tot[...] = tot[...] + b1

    @pl.when(w == NWIN - 1)
    def _():
        z_ref[...] = tot[...]


def _t_body(x_ref, m_ref, z_ref, t_ref):
    m = m_ref[...]
    z = z_ref[...]

    def step(t, acc):
        for u in range(16):
            e = jnp.exp(x_ref[t * 16 + u] - m)
            acc = acc + e / z
        return acc

    t_ref[...] = jax.lax.fori_loop(0, 8, step,
                                   jnp.zeros((RB, NC), jnp.float32))


def _off_body(t_ref, out_ref, u_scr):
    def step(h, acc):
        acc = acc + t_ref[h]
        u_scr[h] = acc
        return acc

    g_tot = jax.lax.fori_loop(0, 128, step, jnp.zeros((NG, 128), jnp.float32))
    rows = []
    a = jnp.zeros((1, 128), jnp.float32)
    for g in range(NG):
        rows.append(a)
        a = a + g_tot[g:g + 1, :]
    off3 = jnp.concatenate(rows, axis=0)

    def step2(h, _):
        out_ref[h] = off3 + u_scr[h]
        return 0

    jax.lax.fori_loop(0, 128, step2, 0)


def _count_body(x_ref, m_ref, z_ref, off_ref, r_ref, a_ref, p_ref):
    m = m_ref[...]
    z = z_ref[...]
    off = off_ref[...]
    r = r_ref[...]
    c_iota = jax.lax.broadcasted_iota(jnp.int32, (RB, NC), 1)
    tail_ok = c_iota != (NC - 1)

    def step(t, carry):
        acc, cnt = carry
        for u in range(16):
            i = t * 16 + u
            e = jnp.exp(x_ref[i] - m)
            p = e / z
            acc = acc + p
            s = acc + off
            valid = jnp.logical_or(i < 32, tail_ok)
            ok = jnp.logical_and(s < r, valid)
            cnt = cnt + ok.astype(jnp.int32)
        return acc, cnt

    _, cnt = jax.lax.fori_loop(
        0, 16, step,
        (jnp.zeros((RB, NC), jnp.float32), jnp.zeros((RB, NC), jnp.int32)))
    k = jnp.sum(cnt, axis=1, keepdims=True)
    a = jnp.minimum(k, V - 1)
    a_ref[...] = a
    c_star = a // 128
    i_star = a - c_star * 128
    c_hit = c_iota == c_star

    def step2(t, sel):
        for u in range(16):
            i = t * 16 + u
            hit = jnp.logical_and(c_hit, i == i_star)
            p = jnp.exp(x_ref[i] - m) / z
            sel = sel + jnp.where(hit, p, jnp.float32(0.0))
        return sel

    sel = jax.lax.fori_loop(0, 8, step2, jnp.zeros((RB, NC), jnp.float32))
    p_ref[...] = jnp.sum(sel, axis=1, keepdims=True)


def kernel(X, r):
    Xt = X.T
    Xp = jnp.pad(X, ((0, 0), (0, NC * 128 - V)), constant_values=-jnp.inf)
    Xr = jnp.transpose(Xp.reshape(B, NC, 128), (2, 0, 1))

    m = pl.pallas_call(
        _max_body,
        grid=(NWIN,),
        in_specs=[pl.BlockSpec((WIN * 8, B), lambda w: (w, 0))],
        out_specs=pl.BlockSpec((1, B), lambda w: (0, 0)),
        out_shape=jax.ShapeDtypeStruct((1, B), jnp.float32),
        scratch_shapes=[pltpu.VMEM((8, 128), jnp.float32)],
    )(Xt)

    z = pl.pallas_call(
        _z_body,
        grid=(NWIN,),
        in_specs=[pl.BlockSpec((WIN * 8, B), lambda w: (w, 0)),
                  pl.BlockSpec((1, B), lambda w: (0, 0))],
        out_specs=pl.BlockSpec((1, B), lambda w: (0, 0)),
        out_shape=jax.ShapeDtypeStruct((1, B), jnp.float32),
        scratch_shapes=[pltpu.VMEM((1, 128), jnp.float32)],
    )(Xt, m)

    mb = m.reshape(B, 1)
    zb = z.reshape(B, 1)

    T = pl.pallas_call(
        _t_body,
        grid=(B // RB,),
        in_specs=[pl.BlockSpec((128, RB, NC), lambda i: (0, i, 0)),
                  pl.BlockSpec((RB, 1), lambda i: (i, 0)),
                  pl.BlockSpec((RB, 1), lambda i: (i, 0))],
        out_specs=pl.BlockSpec((RB, NC), lambda i: (i, 0)),
        out_shape=jax.ShapeDtypeStruct((B, NC), jnp.float32),
    )(Xr, mb, zb)

    Tp = jnp.pad(T, ((0, 0), (0, NG * 128 - NC)))
    Tt = jnp.transpose(Tp.reshape(B, NG, 128), (2, 1, 0))

    Coff = pl.pallas_call(
        _off_body,
        grid=(1,),
        in_specs=[pl.BlockSpec((128, NG, B), lambda i: (0, 0, 0))],
        out_specs=pl.BlockSpec((128, NG, B), lambda i: (0, 0, 0)),
        out_shape=jax.ShapeDtypeStruct((128, NG, B), jnp.float32),
        scratch_shapes=[pltpu.VMEM((128, NG, 128), jnp.float32)],
    )(Tt)

    Coffr = jnp.transpose(Coff, (2, 1, 0)).reshape(B, NG * 128)[:, :NC]
    offset = jnp.concatenate(
        [jnp.zeros((B, 1), jnp.float32), Coffr[:, :NC - 1]], axis=1)

    a, probs = pl.pallas_call(
        _count_body,
        grid=(B // RB,),
        in_specs=[pl.BlockSpec((128, RB, NC), lambda i: (0, i, 0)),
                  pl.BlockSpec((RB, 1), lambda i: (i, 0)),
                  pl.BlockSpec((RB, 1), lambda i: (i, 0)),
                  pl.BlockSpec((RB, NC), lambda i: (i, 0)),
                  pl.BlockSpec((RB, 1), lambda i: (i, 0))],
        out_specs=[pl.BlockSpec((RB, 1), lambda i: (i, 0)),
                   pl.BlockSpec((RB, 1), lambda i: (i, 0))],
        out_shape=[jax.ShapeDtypeStruct((B, 1), jnp.int32),
                   jax.ShapeDtypeStruct((B, 1), jnp.float32)],
    )(Xr, mb, zb, offset, r.reshape(B, 1))

    return a.reshape(B), probs.reshape(B)


# speedup vs baseline: 1.1034x; 1.1034x over previous
import jax
import jax.numpy as jnp
from jax.experimental import pallas as pl
from jax.experimental.pallas import tpu as pltpu

B = 128
V = 100000
NC = 782
NCM = 781
NG = 7
WIN = 962
NWIN = 13
NT = V // 8
RB = 32
NEG_INF = float("-inf")


def _max_body(x_ref, m_ref, macc):
    w = pl.program_id(0)

    @pl.when(w == 0)
    def _():
        macc[...] = jnp.full((8, 128), NEG_INF, jnp.float32)

    valid = 12500 - WIN * w

    def step(t, acc):
        for u in range(13):
            g = t * 13 + u
            sl = x_ref[pl.ds(g * 8, 8), :]
            sl = jnp.where(g < valid, sl, NEG_INF)
            acc = jnp.maximum(acc, sl)
        return acc

    macc[...] = jax.lax.fori_loop(0, WIN // 13, step, macc[...])

    @pl.when(w == NWIN - 1)
    def _():
        m_ref[...] = jnp.max(macc[...], axis=0, keepdims=True)


def _z_body(x_ref, m_ref, z_ref, tot):
    w = pl.program_id(0)

    @pl.when(w == 0)
    def _():
        tot[...] = jnp.zeros((1, 128), jnp.float32)

    m = m_ref[...]
    valid = 12500 - WIN * w

    def step(t, acc):
        for u in range(13):
            g = t * 13 + u
            sl = x_ref[pl.ds(g * 8, 8), :]
            e = jnp.exp(sl - m)
            e = jnp.where(g < valid, e, jnp.float32(0.0))
            acc = acc + e
        return acc

    acc = jax.lax.fori_loop(0, WIN // 13, step,
                            jnp.zeros((8, 128), jnp.float32))
    b4 = acc[0:4, :] + acc[4:8, :]
    b2 = b4[0:2, :] + b4[2:4, :]
    b1 = b2[0:1, :] + b2[1:2, :]
    tot[...] = tot[...] + b1

    @pl.when(w == NWIN - 1)
    def _():
        z_ref[...] = tot[...]


def _t_body(x_ref, m_ref, z_ref, t_ref):
    m = m_ref[...]
    z = z_ref[...]

    def step(t, acc):
        for u in range(16):
            e = jnp.exp(x_ref[t * 16 + u] - m)
            acc = acc + e / z
        return acc

    t_ref[...] = jax.lax.fori_loop(0, 8, step,
                                   jnp.zeros((RB, NC), jnp.float32))


def _off_body(t_ref, out_ref, u_scr):
    def step(h, acc):
        acc = acc + t_ref[h]
        u_scr[h] = acc
        return acc

    g_tot = jax.lax.fori_loop(0, 128, step, jnp.zeros((NG, 128), jnp.float32))
    rows = []
    a = jnp.zeros((1, 128), jnp.float32)
    for g in range(NG):
        rows.append(a)
        a = a + g_tot[g:g + 1, :]
    off3 = jnp.concatenate(rows, axis=0)

    def step2(h, _):
        out_ref[h] = off3 + u_scr[h]
        return 0

    jax.lax.fori_loop(0, 128, step2, 0)


def _count_body(x_ref, m_ref, z_ref, off_ref, r_ref, a_ref, p_ref):
    m = m_ref[...]
    z = z_ref[...]
    off = off_ref[...]
    r = r_ref[...]
    c_iota = jax.lax.broadcasted_iota(jnp.int32, (RB, NC), 1)
    tail_ok = c_iota != (NC - 1)

    def step(t, carry):
        acc, cnt = carry
        for u in range(16):
            i = t * 16 + u
            e = jnp.exp(x_ref[i] - m)
            p = e / z
            acc = acc + p
            s = acc + off
            valid = jnp.logical_or(i < 32, tail_ok)
            ok = jnp.logical_and(s < r, valid)
            cnt = cnt + ok.astype(jnp.int32)
        return acc, cnt

    _, cnt = jax.lax.fori_loop(
        0, 8, step,
        (jnp.zeros((RB, NC), jnp.float32), jnp.zeros((RB, NC), jnp.int32)))
    k = jnp.sum(cnt, axis=1, keepdims=True)
    a = jnp.minimum(k, V - 1)
    a_ref[...] = a
    c_star = a // 128
    i_star = a - c_star * 128
    c_hit = c_iota == c_star

    def step2(t, sel):
        for u in range(16):
            i = t * 16 + u
            hit = jnp.logical_and(c_hit, i == i_star)
            p = jnp.exp(x_ref[i] - m) / z
            sel = sel + jnp.where(hit, p, jnp.float32(0.0))
        return sel

    sel = jax.lax.fori_loop(0, 8, step2, jnp.zeros((RB, NC), jnp.float32))
    p_ref[...] = jnp.sum(sel, axis=1, keepdims=True)


def kernel(X, r):
    Xt = X.T
    Xp = jnp.pad(X, ((0, 0), (0, NC * 128 - V)), constant_values=-jnp.inf)
    Xr = jnp.transpose(Xp.reshape(B, NC, 128), (2, 0, 1))

    m = pl.pallas_call(
        _max_body,
        grid=(NWIN,),
        in_specs=[pl.BlockSpec((WIN * 8, B), lambda w: (w, 0))],
        out_specs=pl.BlockSpec((1, B), lambda w: (0, 0)),
        out_shape=jax.ShapeDtypeStruct((1, B), jnp.float32),
        scratch_shapes=[pltpu.VMEM((8, 128), jnp.float32)],
    )(Xt)

    z = pl.pallas_call(
        _z_body,
        grid=(NWIN,),
        in_specs=[pl.BlockSpec((WIN * 8, B), lambda w: (w, 0)),
                  pl.BlockSpec((1, B), lambda w: (0, 0))],
        out_specs=pl.BlockSpec((1, B), lambda w: (0, 0)),
        out_shape=jax.ShapeDtypeStruct((1, B), jnp.float32),
        scratch_shapes=[pltpu.VMEM((1, 128), jnp.float32)],
    )(Xt, m)

    mb = m.reshape(B, 1)
    zb = z.reshape(B, 1)

    T = pl.pallas_call(
        _t_body,
        grid=(B // RB,),
        in_specs=[pl.BlockSpec((128, RB, NC), lambda i: (0, i, 0)),
                  pl.BlockSpec((RB, 1), lambda i: (i, 0)),
                  pl.BlockSpec((RB, 1), lambda i: (i, 0))],
        out_specs=pl.BlockSpec((RB, NC), lambda i: (i, 0)),
        out_shape=jax.ShapeDtypeStruct((B, NC), jnp.float32),
    )(Xr, mb, zb)

    Tp = jnp.pad(T, ((0, 0), (0, NG * 128 - NC)))
    Tt = jnp.transpose(Tp.reshape(B, NG, 128), (2, 1, 0))

    Coff = pl.pallas_call(
        _off_body,
        grid=(1,),
        in_specs=[pl.BlockSpec((128, NG, B), lambda i: (0, 0, 0))],
        out_specs=pl.BlockSpec((128, NG, B), lambda i: (0, 0, 0)),
        out_shape=jax.ShapeDtypeStruct((128, NG, B), jnp.float32),
        scratch_shapes=[pltpu.VMEM((128, NG, 128), jnp.float32)],
    )(Tt)

    Coffr = jnp.transpose(Coff, (2, 1, 0)).reshape(B, NG * 128)[:, :NC]
    offset = jnp.concatenate(
        [jnp.zeros((B, 1), jnp.float32), Coffr[:, :NC - 1]], axis=1)

    a, probs = pl.pallas_call(
        _count_body,
        grid=(B // RB,),
        in_specs=[pl.BlockSpec((128, RB, NC), lambda i: (0, i, 0)),
                  pl.BlockSpec((RB, 1), lambda i: (i, 0)),
                  pl.BlockSpec((RB, 1), lambda i: (i, 0)),
                  pl.BlockSpec((RB, NC), lambda i: (i, 0)),
                  pl.BlockSpec((RB, 1), lambda i: (i, 0))],
        out_specs=[pl.BlockSpec((RB, 1), lambda i: (i, 0)),
                   pl.BlockSpec((RB, 1), lambda i: (i, 0))],
        out_shape=[jax.ShapeDtypeStruct((B, 1), jnp.int32),
                   jax.ShapeDtypeStruct((B, 1), jnp.float32)],
    )(Xr, mb, zb, offset, r.reshape(B, 1))

    return a.reshape(B), probs.reshape(B)
